# baseline (speedup 1.0000x reference)
import functools

import jax
import jax.numpy as jnp
from jax import lax
from jax.experimental import pallas as pl
from jax.experimental.pallas import tpu as pltpu
from jax.experimental.pallas import tpu_sc as plsc

_L = 16
_NC = 2
_NS = 16
_NW = _NC * _NS
_CHUNK = 128


def _sc_rowgather_body(mus_hbm, sig_hbm, z_hbm, muz_hbm, sigz_hbm,
                       z_v, buf_v, tbuf_v, gs0, gs1, ss0, ss1):
    N, D = mus_hbm.shape
    B = z_hbm.shape[0]
    b_per_w = B // _NW
    n_g = b_per_w // _CHUNK

    wid = lax.axis_index("s") * _NC + lax.axis_index("c")
    base = wid * b_per_w

    for g in range(n_g):
        pltpu.sync_copy(z_hbm.at[pl.ds(base + g * _CHUNK, _CHUNK)], z_v.at[g])

    lane = lax.iota(jnp.int32, _L)
    zero = lane * 0
    cl_idx = [lane + v * _L for v in range(_CHUNK // _L)]

    gsems = (gs0, gs1)
    ssems = (ss0, ss1)
    items = [(src, dst, g)
             for src, dst in ((mus_hbm, muz_hbm), (sig_hbm, sigz_hbm))
             for g in range(n_g)]
    n = len(items)
    pend_g = [None, None]
    pend_s = [None, None]

    def issue_gather(i, b):
        src, _, g = items[i]
        pend_g[b] = pltpu.async_copy(src.at[z_v.at[g]], buf_v.at[b], gsems[b])

    issue_gather(0, 0)
    for i in range(n):
        b = i % 2
        if i + 1 < n:
            b2 = (i + 1) % 2
            if pend_s[b2] is not None:
                pend_s[b2].wait()
                pend_s[b2] = None
            issue_gather(i + 1, b2)
        pend_g[b].wait()
        _, dst, g = items[i]

        @plsc.parallel_loop(0, D, step=1, unroll=4)
        def transpose_step(d, b=b):
            col = zero + d
            for v in range(_CHUNK // _L):
                vals = plsc.load_gather(buf_v.at[b], [cl_idx[v], col])
                tbuf_v[b, d, pl.ds(v * _L, _L)] = vals

        pend_s[b] = pltpu.async_copy(
            tbuf_v.at[b],
            dst.at[:, pl.ds(base + g * _CHUNK, _CHUNK)],
            ssems[b])
    for b in range(2):
        if pend_s[b] is not None:
            pend_s[b].wait()


def kernel(mus, sigmas, z):
    D, N = mus.shape
    B = z.shape[0]
    mus_t = mus.T
    sig_t = sigmas.T
    out = jax.ShapeDtypeStruct((D, B), jnp.float32)
    mesh = plsc.VectorSubcoreMesh(core_axis_name="c", subcore_axis_name="s")
    b_per_w = B // _NW
    n_g = b_per_w // _CHUNK
    k = functools.partial(
        pl.kernel,
        out_type=(out, out),
        mesh=mesh,
        scratch_types=[
            pltpu.VMEM((n_g, _CHUNK), jnp.int32),
            pltpu.VMEM((2, _CHUNK, D), jnp.float32),
            pltpu.VMEM((2, D, _CHUNK), jnp.float32),
            pltpu.SemaphoreType.DMA,
            pltpu.SemaphoreType.DMA,
            pltpu.SemaphoreType.DMA,
            pltpu.SemaphoreType.DMA,
        ],
        compiler_params=pltpu.CompilerParams(needs_layout_passes=False),
    )(_sc_rowgather_body)
    return k(mus_t, sig_t, z)
